# baseline (speedup 1.0000x reference)
import functools

import jax
import jax.numpy as jnp
from jax import lax
from jax.experimental import pallas as pl
from jax.experimental.pallas import tpu as pltpu
from jax.experimental.pallas import tpu_sc as plsc

NC = 2
NS = 16
NW = NC * NS
CHUNK = 128
NBUF = 4


def _sc_mesh():
  return plsc.VectorSubcoreMesh(core_axis_name="c", subcore_axis_name="s")


def _copy_out(sh_ref, out_slice, s, N):
  R = 8 * ((N + 8 * NS - 1) // (8 * NS))
  full = N // R
  rem = N - full * R

  @pl.when(s < full)
  def _():
    pltpu.sync_copy(sh_ref.at[pl.ds(s * R, R)],
                    out_slice.at[pl.ds(s * R, R)])

  if rem:
    @pl.when(s == full)
    def _():
      pltpu.sync_copy(sh_ref.at[pl.ds(full * R, rem)],
                      out_slice.at[pl.ds(full * R, rem)])


def _make_deg_kernel(N, NP, C):
  rows_per_tile_zero = NP // NS

  @functools.partial(
      pl.kernel,
      out_type=jax.ShapeDtypeStruct((NC, N, 16), jnp.float32),
      mesh=_sc_mesh(),
      scratch_types=[
          pltpu.VMEM((C, CHUNK), jnp.int32),
          pltpu.VMEM((CHUNK, 16), jnp.float32),
          pltpu.VMEM_SHARED((NP, 16), jnp.float32),
          pltpu.SemaphoreType.DMA,
      ],
      compiler_params=pltpu.CompilerParams(use_tc_tiling_on_sc=False),
  )
  def deg_kernel(dst_hbm, zeros_hbm, ones_hbm, out_hbm, dstv, ones_v, deg_sh,
                 sem):
    c = lax.axis_index("c")
    s = lax.axis_index("s")
    wid = c * NS + s
    pltpu.sync_copy(dst_hbm.at[wid], dstv)
    pltpu.sync_copy(ones_hbm, ones_v)
    z0 = s * rows_per_tile_zero
    pltpu.sync_copy(zeros_hbm, deg_sh.at[pl.ds(z0, rows_per_tile_zero)])
    plsc.subcore_barrier()

    def fire(j, carry):
      pltpu.async_copy(ones_v, deg_sh.at[dstv.at[j]], sem, add=True)
      return carry

    lax.fori_loop(0, C, fire, 0, unroll=False)

    def drain(j, carry):
      pltpu.make_async_copy(ones_v, deg_sh.at[dstv.at[j]], sem).wait()
      return carry

    lax.fori_loop(0, C, drain, 0, unroll=False)
    plsc.subcore_barrier()
    _copy_out(deg_sh, out_hbm.at[c], s, N)

  return deg_kernel


def _make_edge_kernel(N, NP, C):
  rows_per_tile_zero = NP // NS

  @functools.partial(
      pl.kernel,
      out_type=jax.ShapeDtypeStruct((NC, N, 16), jnp.float32),
      mesh=_sc_mesh(),
      scratch_types=[
          pltpu.VMEM((C, CHUNK), jnp.int32),
          pltpu.VMEM((C, CHUNK), jnp.int32),
          pltpu.VMEM((NBUF, CHUNK, 16), jnp.float32),
          pltpu.VMEM_SHARED((NP, 16), jnp.float32),
          pltpu.SemaphoreType.DMA((NBUF,)),
          pltpu.SemaphoreType.DMA((NBUF,)),
      ],
      compiler_params=pltpu.CompilerParams(use_tc_tiling_on_sc=False),
  )
  def edge_kernel(src_hbm, dst_hbm, y_hbm, zeros_hbm, out_hbm,
                  srcv, dstv, rows_v, acc_sh, sem_g, sem_s):
    c = lax.axis_index("c")
    s = lax.axis_index("s")
    wid = c * NS + s
    pltpu.sync_copy(src_hbm.at[wid], srcv)
    pltpu.sync_copy(dst_hbm.at[wid], dstv)
    z0 = s * rows_per_tile_zero
    pltpu.sync_copy(zeros_hbm, acc_sh.at[pl.ds(z0, rows_per_tile_zero)])
    plsc.subcore_barrier()

    def start_gather(j):
      b = lax.rem(j, NBUF)
      pltpu.async_copy(y_hbm.at[srcv.at[j]], rows_v.at[b], sem_g.at[b])

    def wait_gather(j):
      b = lax.rem(j, NBUF)
      pltpu.make_async_copy(y_hbm.at[srcv.at[j]], rows_v.at[b],
                            sem_g.at[b]).wait()

    def start_scatter(j):
      b = lax.rem(j, NBUF)
      pltpu.async_copy(rows_v.at[b], acc_sh.at[dstv.at[j]], sem_s.at[b],
                       add=True)

    def wait_scatter(j):
      b = lax.rem(j, NBUF)
      pltpu.make_async_copy(rows_v.at[b], acc_sh.at[dstv.at[j]],
                            sem_s.at[b]).wait()

    for j in range(min(NBUF - 1, C)):
      start_gather(jnp.int32(j))

    def body(j, carry):
      wait_gather(j)
      start_scatter(j)

      @pl.when(j >= 1)
      def _():
        wait_scatter(j - 1)

      @pl.when(j + NBUF - 1 < C)
      def _():
        start_gather(j + NBUF - 1)

      return carry

    lax.fori_loop(0, C, body, 0, unroll=False)
    wait_scatter(jnp.int32(C - 1))
    plsc.subcore_barrier()
    _copy_out(acc_sh, out_hbm.at[c], s, N)

  return edge_kernel


def _dense1_body(x_ref, w1_ref, deg_ref, y1_ref, dv_ref):
  dv = lax.rsqrt(1.0 + deg_ref[0] + deg_ref[1])
  xw = jnp.dot(x_ref[...], w1_ref[...], preferred_element_type=jnp.float32)
  dv_ref[...] = dv
  y1_ref[...] = xw * dv


def _dense2_body(y1_ref, acc_ref, dv_ref, b1_ref, w2_ref, y2_ref):
  dv = dv_ref[...]
  h = jnp.maximum(
      dv * (acc_ref[0] + acc_ref[1] + y1_ref[...]) + b1_ref[...], 0.0)
  y2_ref[...] = jnp.dot(
      h, w2_ref[...], preferred_element_type=jnp.float32) * dv


def _dense3_body(y2_ref, acc_ref, dv_ref, b2_ref, z_ref):
  z_ref[...] = dv_ref[...] * (acc_ref[0] + acc_ref[1] + y2_ref[...]) \
      + b2_ref[...]


def kernel(x, edge_index, W1, b1, W2, b2):
  N, d_in = x.shape
  d_h = W1.shape[1]
  d_out = W2.shape[1]
  E = edge_index.shape[1]
  assert d_h == 16 and d_out == 16

  EW = NW * CHUNK
  E_pad = ((E + EW - 1) // EW) * EW
  C = E_pad // EW
  NP = ((N + 127) // 128) * 128
  pad = E_pad - E
  src = jnp.concatenate(
      [edge_index[0].astype(jnp.int32), jnp.zeros((pad,), jnp.int32)])
  dst = jnp.concatenate(
      [edge_index[1].astype(jnp.int32), jnp.full((pad,), N, jnp.int32)])
  src = src.reshape(NW, C, CHUNK)
  dst = dst.reshape(NW, C, CHUNK)

  zeros_tile16 = jnp.zeros((NP // NS, 16), jnp.float32)
  ones_chunk = jnp.ones((CHUNK, 16), jnp.float32)

  deg_kernel = _make_deg_kernel(N, NP, C)
  edge_kernel = _make_edge_kernel(N, NP, C)

  degp = deg_kernel(dst, zeros_tile16, ones_chunk)

  B = 1000
  grid = (N // B,)
  y1, dv = pl.pallas_call(
      _dense1_body,
      grid=grid,
      in_specs=[
          pl.BlockSpec((B, d_in), lambda i: (i, 0)),
          pl.BlockSpec((d_in, d_h), lambda i: (0, 0)),
          pl.BlockSpec((NC, B, 16), lambda i: (0, i, 0)),
      ],
      out_specs=[
          pl.BlockSpec((B, d_h), lambda i: (i, 0)),
          pl.BlockSpec((B, 16), lambda i: (i, 0)),
      ],
      out_shape=[
          jax.ShapeDtypeStruct((N, d_h), jnp.float32),
          jax.ShapeDtypeStruct((N, 16), jnp.float32),
      ],
  )(x, W1, degp)

  acc1 = edge_kernel(src, dst, y1, zeros_tile16)

  y2 = pl.pallas_call(
      _dense2_body,
      grid=grid,
      in_specs=[
          pl.BlockSpec((B, d_h), lambda i: (i, 0)),
          pl.BlockSpec((NC, B, d_h), lambda i: (0, i, 0)),
          pl.BlockSpec((B, 16), lambda i: (i, 0)),
          pl.BlockSpec((1, d_h), lambda i: (0, 0)),
          pl.BlockSpec((d_h, d_out), lambda i: (0, 0)),
      ],
      out_specs=pl.BlockSpec((B, d_out), lambda i: (i, 0)),
      out_shape=jax.ShapeDtypeStruct((N, d_out), jnp.float32),
  )(y1, acc1, dv, b1.reshape(1, d_h), W2)

  acc2 = edge_kernel(src, dst, y2, zeros_tile16)

  z = pl.pallas_call(
      _dense3_body,
      grid=grid,
      in_specs=[
          pl.BlockSpec((B, d_out), lambda i: (i, 0)),
          pl.BlockSpec((NC, B, d_out), lambda i: (0, i, 0)),
          pl.BlockSpec((B, 16), lambda i: (i, 0)),
          pl.BlockSpec((1, d_out), lambda i: (0, 0)),
      ],
      out_specs=pl.BlockSpec((B, d_out), lambda i: (i, 0)),
      out_shape=jax.ShapeDtypeStruct((N, d_out), jnp.float32),
  )(y2, acc2, dv, b2.reshape(1, d_out))

  return z


# speedup vs baseline: 1.0454x; 1.0454x over previous
import functools

import jax
import jax.numpy as jnp
from jax import lax
from jax.experimental import pallas as pl
from jax.experimental.pallas import tpu as pltpu
from jax.experimental.pallas import tpu_sc as plsc

NC = 2
NS = 16
NW = NC * NS
CHUNK = 128
NBUF = 4


def _sc_mesh():
  return plsc.VectorSubcoreMesh(core_axis_name="c", subcore_axis_name="s")


def _copy_out(sh_ref, out_slice, s, N):
  R = 8 * ((N + 8 * NS - 1) // (8 * NS))
  full = N // R
  rem = N - full * R

  @pl.when(s < full)
  def _():
    pltpu.sync_copy(sh_ref.at[pl.ds(s * R, R)],
                    out_slice.at[pl.ds(s * R, R)])

  if rem:
    @pl.when(s == full)
    def _():
      pltpu.sync_copy(sh_ref.at[pl.ds(full * R, rem)],
                      out_slice.at[pl.ds(full * R, rem)])


def _make_deg_kernel(N, NP, C):
  rows_per_tile_zero = NP // NS

  @functools.partial(
      pl.kernel,
      out_type=jax.ShapeDtypeStruct((NC, N, 16), jnp.float32),
      mesh=_sc_mesh(),
      scratch_types=[
          pltpu.VMEM((C, CHUNK), jnp.int32),
          pltpu.VMEM((CHUNK, 16), jnp.float32),
          pltpu.VMEM_SHARED((NP, 16), jnp.float32),
          pltpu.SemaphoreType.DMA,
      ],
      compiler_params=pltpu.CompilerParams(use_tc_tiling_on_sc=False),
  )
  def deg_kernel(dst_hbm, zeros_hbm, ones_hbm, out_hbm, dstv, ones_v, deg_sh,
                 sem):
    c = lax.axis_index("c")
    s = lax.axis_index("s")
    wid = c * NS + s
    pltpu.sync_copy(dst_hbm.at[wid], dstv)
    pltpu.sync_copy(ones_hbm, ones_v)
    z0 = s * rows_per_tile_zero
    pltpu.sync_copy(zeros_hbm, deg_sh.at[pl.ds(z0, rows_per_tile_zero)])
    plsc.subcore_barrier()

    def fire(j, carry):
      pltpu.async_copy(ones_v, deg_sh.at[dstv.at[j]], sem, add=True)
      return carry

    lax.fori_loop(0, C, fire, 0, unroll=False)

    def drain(j, carry):
      pltpu.make_async_copy(ones_v, deg_sh.at[dstv.at[j]], sem).wait()
      return carry

    lax.fori_loop(0, C, drain, 0, unroll=False)
    plsc.subcore_barrier()
    _copy_out(deg_sh, out_hbm.at[c], s, N)

  return deg_kernel


def _make_edge_kernel(N, NP, C):
  rows_per_tile_zero = NP // NS

  @functools.partial(
      pl.kernel,
      out_type=jax.ShapeDtypeStruct((NC, N, 16), jnp.float32),
      mesh=_sc_mesh(),
      scratch_types=[
          pltpu.VMEM((C, CHUNK), jnp.int32),
          pltpu.VMEM((C, CHUNK), jnp.int32),
          pltpu.VMEM((NBUF, CHUNK, 16), jnp.float32),
          pltpu.VMEM_SHARED((NP, 16), jnp.float32),
          pltpu.SemaphoreType.DMA((NBUF,)),
          pltpu.SemaphoreType.DMA((NBUF,)),
      ],
      compiler_params=pltpu.CompilerParams(use_tc_tiling_on_sc=False),
  )
  def edge_kernel(src_hbm, dst_hbm, y_hbm, zeros_hbm, out_hbm,
                  srcv, dstv, rows_v, acc_sh, sem_g, sem_s):
    c = lax.axis_index("c")
    s = lax.axis_index("s")
    wid = c * NS + s
    pltpu.sync_copy(src_hbm.at[wid], srcv)
    pltpu.sync_copy(dst_hbm.at[wid], dstv)
    z0 = s * rows_per_tile_zero
    pltpu.sync_copy(zeros_hbm, acc_sh.at[pl.ds(z0, rows_per_tile_zero)])
    plsc.subcore_barrier()

    def start_gather(j):
      b = lax.rem(j, NBUF)
      pltpu.async_copy(y_hbm.at[srcv.at[j]], rows_v.at[b], sem_g.at[b])

    def wait_gather(j):
      b = lax.rem(j, NBUF)
      pltpu.make_async_copy(y_hbm.at[srcv.at[j]], rows_v.at[b],
                            sem_g.at[b]).wait()

    def start_scatter(j):
      b = lax.rem(j, NBUF)
      pltpu.async_copy(rows_v.at[b], acc_sh.at[dstv.at[j]], sem_s.at[b],
                       add=True)

    def wait_scatter(j):
      b = lax.rem(j, NBUF)
      pltpu.make_async_copy(rows_v.at[b], acc_sh.at[dstv.at[j]],
                            sem_s.at[b]).wait()

    for j in range(min(NBUF - 1, C)):
      start_gather(jnp.int32(j))

    def body(j, carry):
      wait_gather(j)
      start_scatter(j)

      @pl.when(j >= 1)
      def _():
        wait_scatter(j - 1)

      @pl.when(j + NBUF - 1 < C)
      def _():
        start_gather(j + NBUF - 1)

      return carry

    lax.fori_loop(0, C, body, 0, unroll=False)
    wait_scatter(jnp.int32(C - 1))
    plsc.subcore_barrier()
    _copy_out(acc_sh, out_hbm.at[c], s, N)

  return edge_kernel


def _dense1_body(x_ref, w1_ref, deg_ref, y1_ref, dv_ref):
  dv = lax.rsqrt(1.0 + deg_ref[0] + deg_ref[1])
  xw = jnp.dot(x_ref[...], w1_ref[...], preferred_element_type=jnp.float32)
  dv_ref[...] = dv
  y1_ref[...] = xw * dv


def _dense2_body(y1_ref, acc_ref, dv_ref, b1_ref, w2_ref, y2_ref):
  dv = dv_ref[...]
  h = jnp.maximum(
      dv * (acc_ref[0] + acc_ref[1] + y1_ref[...]) + b1_ref[...], 0.0)
  y2_ref[...] = jnp.dot(
      h, w2_ref[...], preferred_element_type=jnp.float32) * dv


def _dense3_body(y2_ref, acc_ref, dv_ref, b2_ref, z_ref):
  z_ref[...] = dv_ref[...] * (acc_ref[0] + acc_ref[1] + y2_ref[...]) \
      + b2_ref[...]


def kernel(x, edge_index, W1, b1, W2, b2):
  N, d_in = x.shape
  d_h = W1.shape[1]
  d_out = W2.shape[1]
  E = edge_index.shape[1]
  assert d_h == 16 and d_out == 16

  EW = NW * CHUNK
  E_pad = ((E + EW - 1) // EW) * EW
  C = E_pad // EW
  NP = ((N + 127) // 128) * 128
  pad = E_pad - E
  src = jnp.concatenate(
      [edge_index[0].astype(jnp.int32), jnp.zeros((pad,), jnp.int32)])
  dst = jnp.concatenate(
      [edge_index[1].astype(jnp.int32), jnp.full((pad,), N, jnp.int32)])
  src = src.reshape(NW, C, CHUNK)
  dst = dst.reshape(NW, C, CHUNK)

  zeros_tile16 = jnp.zeros((NP // NS, 16), jnp.float32)
  ones_chunk = jnp.ones((CHUNK, 16), jnp.float32)

  deg_kernel = _make_deg_kernel(N, NP, C)
  edge_kernel = _make_edge_kernel(N, NP, C)

  degp = deg_kernel(dst, zeros_tile16, ones_chunk)

  B = 2000
  grid = (N // B,)
  y1, dv = pl.pallas_call(
      _dense1_body,
      grid=grid,
      in_specs=[
          pl.BlockSpec((B, d_in), lambda i: (i, 0)),
          pl.BlockSpec((d_in, d_h), lambda i: (0, 0)),
          pl.BlockSpec((NC, B, 16), lambda i: (0, i, 0)),
      ],
      out_specs=[
          pl.BlockSpec((B, d_h), lambda i: (i, 0)),
          pl.BlockSpec((B, 16), lambda i: (i, 0)),
      ],
      out_shape=[
          jax.ShapeDtypeStruct((N, d_h), jnp.float32),
          jax.ShapeDtypeStruct((N, 16), jnp.float32),
      ],
      compiler_params=pltpu.CompilerParams(skip_device_barrier=True),
  )(x, W1, degp)

  acc1 = edge_kernel(src, dst, y1, zeros_tile16)

  y2 = pl.pallas_call(
      _dense2_body,
      grid=grid,
      in_specs=[
          pl.BlockSpec((B, d_h), lambda i: (i, 0)),
          pl.BlockSpec((NC, B, d_h), lambda i: (0, i, 0)),
          pl.BlockSpec((B, 16), lambda i: (i, 0)),
          pl.BlockSpec((1, d_h), lambda i: (0, 0)),
          pl.BlockSpec((d_h, d_out), lambda i: (0, 0)),
      ],
      out_specs=pl.BlockSpec((B, d_out), lambda i: (i, 0)),
      out_shape=jax.ShapeDtypeStruct((N, d_out), jnp.float32),
      compiler_params=pltpu.CompilerParams(skip_device_barrier=True),
  )(y1, acc1, dv, b1.reshape(1, d_h), W2)

  acc2 = edge_kernel(src, dst, y2, zeros_tile16)

  z = pl.pallas_call(
      _dense3_body,
      grid=grid,
      in_specs=[
          pl.BlockSpec((B, d_out), lambda i: (i, 0)),
          pl.BlockSpec((NC, B, d_out), lambda i: (0, i, 0)),
          pl.BlockSpec((B, 16), lambda i: (i, 0)),
          pl.BlockSpec((1, d_out), lambda i: (0, 0)),
      ],
      out_specs=pl.BlockSpec((B, d_out), lambda i: (i, 0)),
      out_shape=jax.ShapeDtypeStruct((N, d_out), jnp.float32),
      compiler_params=pltpu.CompilerParams(skip_device_barrier=True),
  )(y2, acc2, dv, b2.reshape(1, d_out))

  return z


# speedup vs baseline: 1.0999x; 1.0521x over previous
import functools

import jax
import jax.numpy as jnp
from jax import lax
from jax.experimental import pallas as pl
from jax.experimental.pallas import tpu as pltpu
from jax.experimental.pallas import tpu_sc as plsc

NC = 2
NS = 16
NW = NC * NS
CHUNK = 128
NBUF = 8


def _sc_mesh():
  return plsc.VectorSubcoreMesh(core_axis_name="c", subcore_axis_name="s")


def _copy_out(sh_ref, out_slice, s, N):
  R = 8 * ((N + 8 * NS - 1) // (8 * NS))
  full = N // R
  rem = N - full * R

  @pl.when(s < full)
  def _():
    pltpu.sync_copy(sh_ref.at[pl.ds(s * R, R)],
                    out_slice.at[pl.ds(s * R, R)])

  if rem:
    @pl.when(s == full)
    def _():
      pltpu.sync_copy(sh_ref.at[pl.ds(full * R, rem)],
                      out_slice.at[pl.ds(full * R, rem)])


def _make_deg_kernel(N, NP, C):
  rows_per_tile_zero = NP // NS

  @functools.partial(
      pl.kernel,
      out_type=jax.ShapeDtypeStruct((NC, N, 16), jnp.float32),
      mesh=_sc_mesh(),
      scratch_types=[
          pltpu.VMEM((C, CHUNK), jnp.int32),
          pltpu.VMEM((CHUNK, 16), jnp.float32),
          pltpu.VMEM_SHARED((NP, 16), jnp.float32),
          pltpu.SemaphoreType.DMA,
      ],
      compiler_params=pltpu.CompilerParams(use_tc_tiling_on_sc=False),
  )
  def deg_kernel(dst_hbm, zeros_hbm, ones_hbm, out_hbm, dstv, ones_v, deg_sh,
                 sem):
    c = lax.axis_index("c")
    s = lax.axis_index("s")
    wid = c * NS + s
    pltpu.sync_copy(dst_hbm.at[wid], dstv)
    pltpu.sync_copy(ones_hbm, ones_v)
    z0 = s * rows_per_tile_zero
    pltpu.sync_copy(zeros_hbm, deg_sh.at[pl.ds(z0, rows_per_tile_zero)])
    plsc.subcore_barrier()

    def fire(j, carry):
      pltpu.async_copy(ones_v, deg_sh.at[dstv.at[j]], sem, add=True)
      return carry

    lax.fori_loop(0, C, fire, 0, unroll=False)

    def drain(j, carry):
      pltpu.make_async_copy(ones_v, deg_sh.at[dstv.at[j]], sem).wait()
      return carry

    lax.fori_loop(0, C, drain, 0, unroll=False)
    plsc.subcore_barrier()
    _copy_out(deg_sh, out_hbm.at[c], s, N)

  return deg_kernel


def _make_edge_kernel(N, NP, C):
  rows_per_tile_zero = NP // NS

  @functools.partial(
      pl.kernel,
      out_type=jax.ShapeDtypeStruct((NC, N, 16), jnp.float32),
      mesh=_sc_mesh(),
      scratch_types=[
          pltpu.VMEM((C, CHUNK), jnp.int32),
          pltpu.VMEM((C, CHUNK), jnp.int32),
          pltpu.VMEM((NBUF, CHUNK, 16), jnp.float32),
          pltpu.VMEM_SHARED((NP, 16), jnp.float32),
          pltpu.SemaphoreType.DMA((NBUF,)),
          pltpu.SemaphoreType.DMA((NBUF,)),
      ],
      compiler_params=pltpu.CompilerParams(use_tc_tiling_on_sc=False),
  )
  def edge_kernel(src_hbm, dst_hbm, y_hbm, zeros_hbm, out_hbm,
                  srcv, dstv, rows_v, acc_sh, sem_g, sem_s):
    c = lax.axis_index("c")
    s = lax.axis_index("s")
    wid = c * NS + s
    pltpu.sync_copy(src_hbm.at[wid], srcv)
    pltpu.sync_copy(dst_hbm.at[wid], dstv)
    z0 = s * rows_per_tile_zero
    pltpu.sync_copy(zeros_hbm, acc_sh.at[pl.ds(z0, rows_per_tile_zero)])
    plsc.subcore_barrier()

    def start_gather(j):
      b = lax.rem(j, NBUF)
      pltpu.async_copy(y_hbm.at[srcv.at[j]], rows_v.at[b], sem_g.at[b])

    def wait_gather(j):
      b = lax.rem(j, NBUF)
      pltpu.make_async_copy(y_hbm.at[srcv.at[j]], rows_v.at[b],
                            sem_g.at[b]).wait()

    def start_scatter(j):
      b = lax.rem(j, NBUF)
      pltpu.async_copy(rows_v.at[b], acc_sh.at[dstv.at[j]], sem_s.at[b],
                       add=True)

    def wait_scatter(j):
      b = lax.rem(j, NBUF)
      pltpu.make_async_copy(rows_v.at[b], acc_sh.at[dstv.at[j]],
                            sem_s.at[b]).wait()

    for j in range(min(NBUF - 1, C)):
      start_gather(jnp.int32(j))

    def body(j, carry):
      wait_gather(j)
      start_scatter(j)

      @pl.when(j >= 1)
      def _():
        wait_scatter(j - 1)

      @pl.when(j + NBUF - 1 < C)
      def _():
        start_gather(j + NBUF - 1)

      return carry

    lax.fori_loop(0, C, body, 0, unroll=False)
    wait_scatter(jnp.int32(C - 1))
    plsc.subcore_barrier()
    _copy_out(acc_sh, out_hbm.at[c], s, N)

  return edge_kernel


def _dense1_body(x_ref, w1_ref, deg_ref, y1_ref, dv_ref):
  dv = lax.rsqrt(1.0 + deg_ref[0] + deg_ref[1])
  xw = jnp.dot(x_ref[...], w1_ref[...], preferred_element_type=jnp.float32)
  dv_ref[...] = dv
  y1_ref[...] = xw * dv


def _dense2_body(y1_ref, acc_ref, dv_ref, b1_ref, w2_ref, y2_ref):
  dv = dv_ref[...]
  h = jnp.maximum(
      dv * (acc_ref[0] + acc_ref[1] + y1_ref[...]) + b1_ref[...], 0.0)
  y2_ref[...] = jnp.dot(
      h, w2_ref[...], preferred_element_type=jnp.float32) * dv


def _dense3_body(y2_ref, acc_ref, dv_ref, b2_ref, z_ref):
  z_ref[...] = dv_ref[...] * (acc_ref[0] + acc_ref[1] + y2_ref[...]) \
      + b2_ref[...]


def kernel(x, edge_index, W1, b1, W2, b2):
  N, d_in = x.shape
  d_h = W1.shape[1]
  d_out = W2.shape[1]
  E = edge_index.shape[1]
  assert d_h == 16 and d_out == 16

  EW = NW * CHUNK
  E_pad = ((E + EW - 1) // EW) * EW
  C = E_pad // EW
  NP = ((N + 127) // 128) * 128
  pad = E_pad - E
  src = jnp.concatenate(
      [edge_index[0].astype(jnp.int32), jnp.zeros((pad,), jnp.int32)])
  dst = jnp.concatenate(
      [edge_index[1].astype(jnp.int32), jnp.full((pad,), N, jnp.int32)])
  src = src.reshape(NW, C, CHUNK)
  dst = dst.reshape(NW, C, CHUNK)

  zeros_tile16 = jnp.zeros((NP // NS, 16), jnp.float32)
  ones_chunk = jnp.ones((CHUNK, 16), jnp.float32)

  deg_kernel = _make_deg_kernel(N, NP, C)
  edge_kernel = _make_edge_kernel(N, NP, C)

  degp = deg_kernel(dst, zeros_tile16, ones_chunk)

  B = 2000
  grid = (N // B,)
  y1, dv = pl.pallas_call(
      _dense1_body,
      grid=grid,
      in_specs=[
          pl.BlockSpec((B, d_in), lambda i: (i, 0)),
          pl.BlockSpec((d_in, d_h), lambda i: (0, 0)),
          pl.BlockSpec((NC, B, 16), lambda i: (0, i, 0)),
      ],
      out_specs=[
          pl.BlockSpec((B, d_h), lambda i: (i, 0)),
          pl.BlockSpec((B, 16), lambda i: (i, 0)),
      ],
      out_shape=[
          jax.ShapeDtypeStruct((N, d_h), jnp.float32),
          jax.ShapeDtypeStruct((N, 16), jnp.float32),
      ],
      compiler_params=pltpu.CompilerParams(skip_device_barrier=True),
  )(x, W1, degp)

  acc1 = edge_kernel(src, dst, y1, zeros_tile16)

  y2 = pl.pallas_call(
      _dense2_body,
      grid=grid,
      in_specs=[
          pl.BlockSpec((B, d_h), lambda i: (i, 0)),
          pl.BlockSpec((NC, B, d_h), lambda i: (0, i, 0)),
          pl.BlockSpec((B, 16), lambda i: (i, 0)),
          pl.BlockSpec((1, d_h), lambda i: (0, 0)),
          pl.BlockSpec((d_h, d_out), lambda i: (0, 0)),
      ],
      out_specs=pl.BlockSpec((B, d_out), lambda i: (i, 0)),
      out_shape=jax.ShapeDtypeStruct((N, d_out), jnp.float32),
      compiler_params=pltpu.CompilerParams(skip_device_barrier=True),
  )(y1, acc1, dv, b1.reshape(1, d_h), W2)

  acc2 = edge_kernel(src, dst, y2, zeros_tile16)

  z = pl.pallas_call(
      _dense3_body,
      grid=grid,
      in_specs=[
          pl.BlockSpec((B, d_out), lambda i: (i, 0)),
          pl.BlockSpec((NC, B, d_out), lambda i: (0, i, 0)),
          pl.BlockSpec((B, 16), lambda i: (i, 0)),
          pl.BlockSpec((1, d_out), lambda i: (0, 0)),
      ],
      out_specs=pl.BlockSpec((B, d_out), lambda i: (i, 0)),
      out_shape=jax.ShapeDtypeStruct((N, d_out), jnp.float32),
      compiler_params=pltpu.CompilerParams(skip_device_barrier=True),
  )(y2, acc2, dv, b2.reshape(1, d_out))

  return z


# speedup vs baseline: 1.2856x; 1.1689x over previous
import functools

import jax
import jax.numpy as jnp
from jax import lax
from jax.experimental import pallas as pl
from jax.experimental.pallas import tpu as pltpu
from jax.experimental.pallas import tpu_sc as plsc

NC = 2
NS = 16
CHUNK = 128
NBUF = 8
FAST_SHARE = 0.61


def _sc_mesh():
  return plsc.VectorSubcoreMesh(core_axis_name="c", subcore_axis_name="s")


def _copy_out(sh_ref, out_slice, s, N):
  R = 8 * ((N + 8 * NS - 1) // (8 * NS))
  full = N // R
  rem = N - full * R

  @pl.when(s < full)
  def _():
    pltpu.sync_copy(sh_ref.at[pl.ds(s * R, R)],
                    out_slice.at[pl.ds(s * R, R)])

  if rem:
    @pl.when(s == full)
    def _():
      pltpu.sync_copy(sh_ref.at[pl.ds(full * R, rem)],
                      out_slice.at[pl.ds(full * R, rem)])


def _stage_idx(idx0_hbm, idx1_hbm, buf, c, s, C0, C1):
  @pl.when(c == 0)
  def _():
    pltpu.sync_copy(idx0_hbm.at[s], buf.at[pl.ds(0, C0)])

  @pl.when(c == 1)
  def _():
    pltpu.sync_copy(idx1_hbm.at[s], buf.at[pl.ds(0, C1)])


def _make_deg_kernel(N, NP, C0, C1):
  rows_per_tile_zero = NP // NS

  @functools.partial(
      pl.kernel,
      out_type=jax.ShapeDtypeStruct((NC, N, 16), jnp.float32),
      mesh=_sc_mesh(),
      scratch_types=[
          pltpu.VMEM((C0, CHUNK), jnp.int32),
          pltpu.VMEM((CHUNK, 16), jnp.float32),
          pltpu.VMEM_SHARED((NP, 16), jnp.float32),
          pltpu.SemaphoreType.DMA,
      ],
      compiler_params=pltpu.CompilerParams(use_tc_tiling_on_sc=False),
  )
  def deg_kernel(dst0_hbm, dst1_hbm, zeros_hbm, ones_hbm, out_hbm,
                 dstv, ones_v, deg_sh, sem):
    c = lax.axis_index("c")
    s = lax.axis_index("s")
    CC = jnp.where(c == 0, C0, C1)
    _stage_idx(dst0_hbm, dst1_hbm, dstv, c, s, C0, C1)
    pltpu.sync_copy(ones_hbm, ones_v)
    z0 = s * rows_per_tile_zero
    pltpu.sync_copy(zeros_hbm, deg_sh.at[pl.ds(z0, rows_per_tile_zero)])
    plsc.subcore_barrier()

    def fire(j, carry):
      pltpu.async_copy(ones_v, deg_sh.at[dstv.at[j]], sem, add=True)
      return carry

    lax.fori_loop(0, CC, fire, 0, unroll=False)

    def drain(j, carry):
      pltpu.make_async_copy(ones_v, deg_sh.at[dstv.at[j]], sem).wait()
      return carry

    lax.fori_loop(0, CC, drain, 0, unroll=False)
    plsc.subcore_barrier()
    _copy_out(deg_sh, out_hbm.at[c], s, N)

  return deg_kernel


def _make_edge_kernel(N, NP, C0, C1):
  rows_per_tile_zero = NP // NS

  @functools.partial(
      pl.kernel,
      out_type=jax.ShapeDtypeStruct((NC, N, 16), jnp.float32),
      mesh=_sc_mesh(),
      scratch_types=[
          pltpu.VMEM((C0, CHUNK), jnp.int32),
          pltpu.VMEM((C0, CHUNK), jnp.int32),
          pltpu.VMEM((NBUF, CHUNK, 16), jnp.float32),
          pltpu.VMEM_SHARED((NP, 16), jnp.float32),
          pltpu.SemaphoreType.DMA((NBUF,)),
          pltpu.SemaphoreType.DMA((NBUF,)),
      ],
      compiler_params=pltpu.CompilerParams(use_tc_tiling_on_sc=False),
  )
  def edge_kernel(src0_hbm, src1_hbm, dst0_hbm, dst1_hbm, y_hbm, zeros_hbm,
                  out_hbm, srcv, dstv, rows_v, acc_sh, sem_g, sem_s):
    c = lax.axis_index("c")
    s = lax.axis_index("s")
    CC = jnp.where(c == 0, C0, C1)
    _stage_idx(src0_hbm, src1_hbm, srcv, c, s, C0, C1)
    _stage_idx(dst0_hbm, dst1_hbm, dstv, c, s, C0, C1)
    z0 = s * rows_per_tile_zero
    pltpu.sync_copy(zeros_hbm, acc_sh.at[pl.ds(z0, rows_per_tile_zero)])
    plsc.subcore_barrier()

    def start_gather(j):
      b = lax.rem(j, NBUF)
      pltpu.async_copy(y_hbm.at[srcv.at[j]], rows_v.at[b], sem_g.at[b])

    def wait_gather(j):
      b = lax.rem(j, NBUF)
      pltpu.make_async_copy(y_hbm.at[srcv.at[j]], rows_v.at[b],
                            sem_g.at[b]).wait()

    def start_scatter(j):
      b = lax.rem(j, NBUF)
      pltpu.async_copy(rows_v.at[b], acc_sh.at[dstv.at[j]], sem_s.at[b],
                       add=True)

    def wait_scatter(j):
      b = lax.rem(j, NBUF)
      pltpu.make_async_copy(rows_v.at[b], acc_sh.at[dstv.at[j]],
                            sem_s.at[b]).wait()

    for j in range(NBUF - 1):
      jj = jnp.int32(j)

      @pl.when(jj < CC)
      def _():
        start_gather(jj)

    def body(j, carry):
      wait_gather(j)
      start_scatter(j)

      @pl.when(j >= 1)
      def _():
        wait_scatter(j - 1)

      @pl.when(j + NBUF - 1 < CC)
      def _():
        start_gather(j + NBUF - 1)

      return carry

    lax.fori_loop(0, CC, body, 0, unroll=False)
    wait_scatter(CC - 1)
    plsc.subcore_barrier()
    _copy_out(acc_sh, out_hbm.at[c], s, N)

  return edge_kernel


def _dense1_body(x_ref, w1_ref, deg_ref, y1_ref, dv_ref):
  dv = lax.rsqrt(1.0 + deg_ref[0] + deg_ref[1])
  xw = jnp.dot(x_ref[...], w1_ref[...], preferred_element_type=jnp.float32)
  dv_ref[...] = dv
  y1_ref[...] = xw * dv


def _dense2_body(y1_ref, acc_ref, dv_ref, b1_ref, w2_ref, y2_ref):
  dv = dv_ref[...]
  h = jnp.maximum(
      dv * (acc_ref[0] + acc_ref[1] + y1_ref[...]) + b1_ref[...], 0.0)
  y2_ref[...] = jnp.dot(
      h, w2_ref[...], preferred_element_type=jnp.float32) * dv


def _dense3_body(y2_ref, acc_ref, dv_ref, b2_ref, z_ref):
  z_ref[...] = dv_ref[...] * (acc_ref[0] + acc_ref[1] + y2_ref[...]) \
      + b2_ref[...]


def kernel(x, edge_index, W1, b1, W2, b2):
  N, d_in = x.shape
  d_h = W1.shape[1]
  d_out = W2.shape[1]
  E = edge_index.shape[1]
  assert d_h == 16 and d_out == 16

  PW = NS * CHUNK
  C0 = int(round(E * FAST_SHARE / PW))
  C1 = -(-(E - C0 * PW) // PW)
  E_cap = (C0 + C1) * PW
  NP = ((N + 127) // 128) * 128
  pad = E_cap - E
  src = jnp.concatenate(
      [edge_index[0].astype(jnp.int32), jnp.zeros((pad,), jnp.int32)])
  dst = jnp.concatenate(
      [edge_index[1].astype(jnp.int32), jnp.full((pad,), N, jnp.int32)])
  split = C0 * PW
  src0 = src[:split].reshape(NS, C0, CHUNK)
  src1 = src[split:].reshape(NS, C1, CHUNK)
  dst0 = dst[:split].reshape(NS, C0, CHUNK)
  dst1 = dst[split:].reshape(NS, C1, CHUNK)

  zeros_tile16 = jnp.zeros((NP // NS, 16), jnp.float32)
  ones_chunk = jnp.ones((CHUNK, 16), jnp.float32)

  deg_kernel = _make_deg_kernel(N, NP, C0, C1)
  edge_kernel = _make_edge_kernel(N, NP, C0, C1)

  degp = deg_kernel(dst0, dst1, zeros_tile16, ones_chunk)

  B = 2000
  grid = (N // B,)
  y1, dv = pl.pallas_call(
      _dense1_body,
      grid=grid,
      in_specs=[
          pl.BlockSpec((B, d_in), lambda i: (i, 0)),
          pl.BlockSpec((d_in, d_h), lambda i: (0, 0)),
          pl.BlockSpec((NC, B, 16), lambda i: (0, i, 0)),
      ],
      out_specs=[
          pl.BlockSpec((B, d_h), lambda i: (i, 0)),
          pl.BlockSpec((B, 16), lambda i: (i, 0)),
      ],
      out_shape=[
          jax.ShapeDtypeStruct((N, d_h), jnp.float32),
          jax.ShapeDtypeStruct((N, 16), jnp.float32),
      ],
      compiler_params=pltpu.CompilerParams(skip_device_barrier=True),
  )(x, W1, degp)

  acc1 = edge_kernel(src0, src1, dst0, dst1, y1, zeros_tile16)

  y2 = pl.pallas_call(
      _dense2_body,
      grid=grid,
      in_specs=[
          pl.BlockSpec((B, d_h), lambda i: (i, 0)),
          pl.BlockSpec((NC, B, d_h), lambda i: (0, i, 0)),
          pl.BlockSpec((B, 16), lambda i: (i, 0)),
          pl.BlockSpec((1, d_h), lambda i: (0, 0)),
          pl.BlockSpec((d_h, d_out), lambda i: (0, 0)),
      ],
      out_specs=pl.BlockSpec((B, d_out), lambda i: (i, 0)),
      out_shape=jax.ShapeDtypeStruct((N, d_out), jnp.float32),
      compiler_params=pltpu.CompilerParams(skip_device_barrier=True),
  )(y1, acc1, dv, b1.reshape(1, d_h), W2)

  acc2 = edge_kernel(src0, src1, dst0, dst1, y2, zeros_tile16)

  z = pl.pallas_call(
      _dense3_body,
      grid=grid,
      in_specs=[
          pl.BlockSpec((B, d_out), lambda i: (i, 0)),
          pl.BlockSpec((NC, B, d_out), lambda i: (0, i, 0)),
          pl.BlockSpec((B, 16), lambda i: (i, 0)),
          pl.BlockSpec((1, d_out), lambda i: (0, 0)),
      ],
      out_specs=pl.BlockSpec((B, d_out), lambda i: (i, 0)),
      out_shape=jax.ShapeDtypeStruct((N, d_out), jnp.float32),
      compiler_params=pltpu.CompilerParams(skip_device_barrier=True),
  )(y2, acc2, dv, b2.reshape(1, d_out))

  return z


# speedup vs baseline: 2.0470x; 1.5922x over previous
import functools

import jax
import jax.numpy as jnp
from jax import lax
from jax.experimental import pallas as pl
from jax.experimental.pallas import tpu as pltpu
from jax.experimental.pallas import tpu_sc as plsc

NC = 2
NS = 16
CHUNK = 128
NBUF = 8
FAST_SHARE = 0.58


def _sc_mesh():
  return plsc.VectorSubcoreMesh(core_axis_name="c", subcore_axis_name="s")


def _copy_out(sh_ref, out_slice, s, N):
  R = 8 * ((N + 8 * NS - 1) // (8 * NS))
  full = N // R
  rem = N - full * R

  @pl.when(s < full)
  def _():
    pltpu.sync_copy(sh_ref.at[pl.ds(s * R, R)],
                    out_slice.at[pl.ds(s * R, R)])

  if rem:
    @pl.when(s == full)
    def _():
      pltpu.sync_copy(sh_ref.at[pl.ds(full * R, rem)],
                      out_slice.at[pl.ds(full * R, rem)])


def _plan(E, share):
  K = -(-E // CHUNK)
  C0 = max(8, 8 * round(K * share / (NS * 8)))
  K1 = K - NS * C0
  assert K1 > 0
  C1 = 8 * (K1 // ((NS - 1) * 8))
  tail = K1 - (NS - 1) * C1
  assert 0 <= tail and C1 >= 8 and tail <= C0 and C1 <= C0
  return K, C0, C1, tail


def _stage_idx(e3_row, buf, c, s, C0, C1, tail):
  off1 = NS * C0

  @pl.when(c == 0)
  def _():
    pltpu.sync_copy(e3_row.at[pl.ds(s * C0, C0)], buf.at[pl.ds(0, C0)])

  @pl.when((c == 1) & (s < NS - 1))
  def _():
    pltpu.sync_copy(e3_row.at[pl.ds(off1 + s * C1, C1)],
                    buf.at[pl.ds(0, C1)])

  if tail:
    @pl.when((c == 1) & (s == NS - 1))
    def _():
      pltpu.sync_copy(e3_row.at[pl.ds(off1 + (NS - 1) * C1, tail)],
                      buf.at[pl.ds(0, tail)])


def _tile_chunks(c, s, C0, C1, tail):
  return jnp.where(c == 0, C0, jnp.where(s < NS - 1, C1, tail))


def _make_deg_kernel(N, NP, C0, C1, tail):
  rows_per_tile_zero = NP // NS

  @functools.partial(
      pl.kernel,
      out_type=jax.ShapeDtypeStruct((NC, N, 16), jnp.float32),
      mesh=_sc_mesh(),
      scratch_types=[
          pltpu.VMEM((C0, CHUNK), jnp.int32),
          pltpu.VMEM((CHUNK, 16), jnp.float32),
          pltpu.VMEM_SHARED((NP, 16), jnp.float32),
          pltpu.SemaphoreType.DMA,
      ],
      compiler_params=pltpu.CompilerParams(use_tc_tiling_on_sc=False),
  )
  def deg_kernel(e3_hbm, zeros_hbm, ones_hbm, out_hbm,
                 dstv, ones_v, deg_sh, sem):
    c = lax.axis_index("c")
    s = lax.axis_index("s")
    CC = _tile_chunks(c, s, C0, C1, tail)
    _stage_idx(e3_hbm.at[1], dstv, c, s, C0, C1, tail)
    pltpu.sync_copy(ones_hbm, ones_v)
    z0 = s * rows_per_tile_zero
    pltpu.sync_copy(zeros_hbm, deg_sh.at[pl.ds(z0, rows_per_tile_zero)])
    plsc.subcore_barrier()

    def fire(j, carry):
      pltpu.async_copy(ones_v, deg_sh.at[dstv.at[j]], sem, add=True)
      return carry

    lax.fori_loop(0, CC, fire, 0, unroll=False)

    def drain(j, carry):
      pltpu.make_async_copy(ones_v, deg_sh.at[dstv.at[j]], sem).wait()
      return carry

    lax.fori_loop(0, CC, drain, 0, unroll=False)
    plsc.subcore_barrier()
    _copy_out(deg_sh, out_hbm.at[c], s, N)

  return deg_kernel


def _make_edge_kernel(N, NP, C0, C1, tail):
  rows_per_tile_zero = NP // NS

  @functools.partial(
      pl.kernel,
      out_type=jax.ShapeDtypeStruct((NC, N, 16), jnp.float32),
      mesh=_sc_mesh(),
      scratch_types=[
          pltpu.VMEM((C0, CHUNK), jnp.int32),
          pltpu.VMEM((C0, CHUNK), jnp.int32),
          pltpu.VMEM((NBUF, CHUNK, 16), jnp.float32),
          pltpu.VMEM_SHARED((NP, 16), jnp.float32),
          pltpu.SemaphoreType.DMA((NBUF,)),
          pltpu.SemaphoreType.DMA((NBUF,)),
      ],
      compiler_params=pltpu.CompilerParams(use_tc_tiling_on_sc=False),
  )
  def edge_kernel(e3_hbm, y_hbm, zeros_hbm,
                  out_hbm, srcv, dstv, rows_v, acc_sh, sem_g, sem_s):
    c = lax.axis_index("c")
    s = lax.axis_index("s")
    CC = _tile_chunks(c, s, C0, C1, tail)
    _stage_idx(e3_hbm.at[0], srcv, c, s, C0, C1, tail)
    _stage_idx(e3_hbm.at[1], dstv, c, s, C0, C1, tail)
    z0 = s * rows_per_tile_zero
    pltpu.sync_copy(zeros_hbm, acc_sh.at[pl.ds(z0, rows_per_tile_zero)])
    plsc.subcore_barrier()

    def start_gather(j):
      b = lax.rem(j, NBUF)
      pltpu.async_copy(y_hbm.at[srcv.at[j]], rows_v.at[b], sem_g.at[b])

    def wait_gather(j):
      b = lax.rem(j, NBUF)
      pltpu.make_async_copy(y_hbm.at[srcv.at[j]], rows_v.at[b],
                            sem_g.at[b]).wait()

    def start_scatter(j):
      b = lax.rem(j, NBUF)
      pltpu.async_copy(rows_v.at[b], acc_sh.at[dstv.at[j]], sem_s.at[b],
                       add=True)

    def wait_scatter(j):
      b = lax.rem(j, NBUF)
      pltpu.make_async_copy(rows_v.at[b], acc_sh.at[dstv.at[j]],
                            sem_s.at[b]).wait()

    for j in range(NBUF - 1):
      jj = jnp.int32(j)

      @pl.when(jj < CC)
      def _():
        start_gather(jj)

    def body(j, carry):
      wait_gather(j)
      start_scatter(j)

      @pl.when(j >= 1)
      def _():
        wait_scatter(j - 1)

      @pl.when(j + NBUF - 1 < CC)
      def _():
        start_gather(j + NBUF - 1)

      return carry

    lax.fori_loop(0, CC, body, 0, unroll=False)
    wait_scatter(CC - 1)
    plsc.subcore_barrier()
    _copy_out(acc_sh, out_hbm.at[c], s, N)

  return edge_kernel


def _pdense1a_body(x_ref, w1b_ref, xw_ref):
  xw_ref[...] = jnp.dot(
      x_ref[...], w1b_ref[...], preferred_element_type=jnp.float32)


def _pdense1b_body(xw_ref, deg_ref, y1_ref, dv_ref):
  dv = lax.rsqrt(1.0 + deg_ref[0] + deg_ref[1])
  dv_ref[...] = dv
  y1_ref[...] = xw_ref[...] * dv


def _pdense2_body(y1_ref, acc_ref, dv_ref, b1_ref, w2b_ref, y2_ref):
  dv = dv_ref[...]
  h = jnp.maximum(
      dv * (acc_ref[0] + acc_ref[1] + y1_ref[...]) + b1_ref[...], 0.0)
  y2_ref[...] = jnp.dot(
      h, w2b_ref[...], preferred_element_type=jnp.float32) * dv


def _pdense3_body(y2_ref, acc_ref, dv_ref, b2_ref, z_ref):
  z_ref[...] = dv_ref[...] * (acc_ref[0] + acc_ref[1] + y2_ref[...]) \
      + b2_ref[...]


def kernel(x, edge_index, W1, b1, W2, b2):
  N, d_in = x.shape
  d_h = W1.shape[1]
  d_out = W2.shape[1]
  E = edge_index.shape[1]
  assert d_h == 16 and d_out == 16

  K, C0, C1, tail = _plan(E, FAST_SHARE)
  NP = ((N + 127) // 128) * 128
  pad = K * CHUNK - E
  ei = edge_index.astype(jnp.int32)
  if pad:
    ei = jnp.concatenate(
        [ei, jnp.stack([jnp.zeros((pad,), jnp.int32),
                        jnp.full((pad,), N, jnp.int32)])], axis=1)
  e3 = ei.reshape(2, K, CHUNK)

  zeros_tile16 = jnp.zeros((NP // NS, 16), jnp.float32)
  ones_chunk = jnp.ones((CHUNK, 16), jnp.float32)

  deg_kernel = _make_deg_kernel(N, NP, C0, C1, tail)
  edge_kernel = _make_edge_kernel(N, NP, C0, C1, tail)

  NR = N // 8
  xp = x.reshape(NR, 8 * d_in)
  w1b = jnp.kron(jnp.eye(8, dtype=jnp.float32), W1)
  w2b = jnp.kron(jnp.eye(8, dtype=jnp.float32), W2)
  b1p = jnp.tile(b1, 8).reshape(1, 8 * d_h)
  b2p = jnp.tile(b2, 8).reshape(1, 8 * d_out)
  full = lambda *shape: pl.BlockSpec(shape, lambda: tuple(0 for _ in shape))
  cp = pltpu.CompilerParams(skip_device_barrier=True)

  xw1 = pl.pallas_call(
      _pdense1a_body,
      in_specs=[full(NR, 8 * d_in), full(8 * d_in, 8 * d_h)],
      out_specs=full(NR, 8 * d_h),
      out_shape=jax.ShapeDtypeStruct((NR, 8 * d_h), jnp.float32),
      compiler_params=cp,
  )(xp, w1b)

  degp = deg_kernel(e3, zeros_tile16, ones_chunk).reshape(NC, NR, 128)

  y1, dv = pl.pallas_call(
      _pdense1b_body,
      in_specs=[full(NR, 128), full(NC, NR, 128)],
      out_specs=[full(NR, 128), full(NR, 128)],
      out_shape=[
          jax.ShapeDtypeStruct((NR, 128), jnp.float32),
          jax.ShapeDtypeStruct((NR, 128), jnp.float32),
      ],
      compiler_params=cp,
  )(xw1, degp)

  acc1 = edge_kernel(e3, y1.reshape(N, 16), zeros_tile16)

  y2 = pl.pallas_call(
      _pdense2_body,
      in_specs=[full(NR, 128), full(NC, NR, 128), full(NR, 128),
                full(1, 128), full(128, 128)],
      out_specs=full(NR, 128),
      out_shape=jax.ShapeDtypeStruct((NR, 128), jnp.float32),
      compiler_params=cp,
  )(y1, acc1.reshape(NC, NR, 128), dv, b1p, w2b)

  acc2 = edge_kernel(e3, y2.reshape(N, 16), zeros_tile16)

  z = pl.pallas_call(
      _pdense3_body,
      in_specs=[full(NR, 128), full(NC, NR, 128), full(NR, 128),
                full(1, 128)],
      out_specs=full(NR, 128),
      out_shape=jax.ShapeDtypeStruct((NR, 128), jnp.float32),
      compiler_params=cp,
  )(y2, acc2.reshape(NC, NR, 128), dv, b2p)
  z = z.reshape(N, d_out)

  return z
